# baseline (speedup 1.0000x reference)
import dataclasses
import functools

import jax
import jax.numpy as jnp
from jax import lax
from jax.experimental import pallas as pl
from jax.experimental.pallas import tpu as pltpu
from jax.experimental.pallas import tpu_sc as plsc

B = 16384
D = 16
NC, NS = 2, 16
NW = NC * NS
BPW = B // NW
NBUF = 4
LANES = 128


def _sc_gather(uids, iids, gu_t, gi_t, mu_t, mi_t):
    mesh = plsc.VectorSubcoreMesh(
        core_axis_name="c", subcore_axis_name="s",
        num_cores=NC, num_subcores=NS)

    ring_types = [pltpu.VMEM((D, LANES), jnp.float32)
                  for _ in range(4 * NBUF)]

    cp = pltpu.CompilerParams()
    if "needs_layout_passes" in pltpu.CompilerParams.__dataclass_fields__:
        cp = dataclasses.replace(cp, needs_layout_passes=False)

    @functools.partial(
        pl.kernel,
        out_type=(jax.ShapeDtypeStruct((D, B), jnp.float32),
                  jax.ShapeDtypeStruct((2 * D, B), jnp.float32)),
        mesh=mesh,
        compiler_params=cp,
        scratch_types=(
            [pltpu.SMEM((BPW,), jnp.int32),
             pltpu.SMEM((BPW,), jnp.int32),
             pltpu.VMEM((BPW,), jnp.int32),
             pltpu.VMEM((BPW,), jnp.int32)]
            + ring_types
            + [pltpu.VMEM((D, BPW), jnp.float32),
               pltpu.VMEM((2 * D, BPW), jnp.float32)]
            + [pltpu.SemaphoreType.DMA for _ in range(NBUF)]
        ),
    )
    def k(t_uids, t_iids, t_gu, t_gi, t_mu, t_mi, o_g, o_mlp, *scratch):
        su, si, vu, vi = scratch[0], scratch[1], scratch[2], scratch[3]
        rings = scratch[4:4 + 4 * NBUF]
        g_st = scratch[4 + 4 * NBUF]
        m_st = scratch[5 + 4 * NBUF]
        sems = scratch[6 + 4 * NBUF:]
        tabs = (t_gu, t_gi, t_mu, t_mi)

        wid = lax.axis_index("s") * NC + lax.axis_index("c")
        base = wid * BPW
        pltpu.sync_copy(t_uids.at[pl.ds(base, BPW)], vu)
        pltpu.sync_copy(t_iids.at[pl.ds(base, BPW)], vi)

        iota16 = jax.lax.iota(jnp.int32, D)

        def spill(c, _):
            off = pl.multiple_of(c * 16, 8)
            uc = vu[pl.ds(off, 16)]
            ic = vi[pl.ds(off, 16)]
            for j in range(16):
                su[c * 16 + j] = jnp.sum(jnp.where(iota16 == j, uc, 0))
                si[c * 16 + j] = jnp.sum(jnp.where(iota16 == j, ic, 0))
            return 0

        lax.fori_loop(0, BPW // 16, spill, 0)

        def fire(i, slot):
            ub = pl.multiple_of((su[i] >> 7) * LANES, LANES)
            ib = pl.multiple_of((si[i] >> 7) * LANES, LANES)
            offs = (ub, ib, ub, ib)
            for t in range(4):
                pltpu.async_copy(tabs[t].at[:, pl.ds(offs[t], LANES)],
                                 rings[t * NBUF + slot], sems[slot])

        for slot in range(NBUF):
            fire(slot, slot)

        iota = iota16

        def group(gidx, _):
            for slot in range(NBUF):
                i = gidx * NBUF + slot
                for t in range(4):
                    pltpu.make_async_copy(
                        tabs[t].at[:, pl.ds(0, LANES)],
                        rings[t * NBUF + slot], sems[slot]).wait()
                lu = jnp.full((D,), su[i] & (LANES - 1), jnp.int32)
                li = jnp.full((D,), si[i] & (LANES - 1), jnp.int32)
                gu = plsc.load_gather(rings[0 * NBUF + slot], [iota, lu])
                gi = plsc.load_gather(rings[1 * NBUF + slot], [iota, li])
                mu = plsc.load_gather(rings[2 * NBUF + slot], [iota, lu])
                mi = plsc.load_gather(rings[3 * NBUF + slot], [iota, li])
                col = jnp.full((D,), i, jnp.int32)
                plsc.store_scatter(g_st, [iota, col], gu * gi)
                plsc.store_scatter(m_st, [iota, col], mu)
                plsc.store_scatter(m_st, [iota + D, col], mi)
                nxt = jnp.minimum(i + NBUF, BPW - 1)
                fire(nxt, slot)
            return 0

        lax.fori_loop(0, BPW // NBUF, group, 0)
        for slot in range(NBUF):
            for t in range(4):
                pltpu.make_async_copy(
                    tabs[t].at[:, pl.ds(0, LANES)],
                    rings[t * NBUF + slot], sems[slot]).wait()
        pltpu.sync_copy(g_st, o_g.at[:, pl.ds(base, BPW)])
        pltpu.sync_copy(m_st, o_mlp.at[:, pl.ds(base, BPW)])

    return k(uids, iids, gu_t, gi_t, mu_t, mi_t)


def _tc_mlp_body(g, m, w1t, b1, w2t, b2, wgt, wht, bpr, out_ref):
    f32 = jnp.float32
    h1 = jnp.dot(w1t[...], m[...], preferred_element_type=f32)
    h1 = jnp.maximum(h1 + b1[...][:, None], 0.0)
    h2 = jnp.dot(w2t[...], h1, preferred_element_type=f32)
    h2 = jnp.maximum(h2 + b2[...][:, None], 0.0)
    o = jnp.dot(wgt[...], g[...], preferred_element_type=f32)
    o = o + jnp.dot(wht[...], h2, preferred_element_type=f32)
    out_ref[...] = o + bpr[...][:, None]


def _tc_mlp(g, m, W1t, b1, W2t, b2, Wgt, Wht, bp):
    return pl.pallas_call(
        _tc_mlp_body,
        out_shape=jax.ShapeDtypeStruct((1, B), jnp.float32),
    )(g, m, W1t, b1, W2t, b2, Wgt, Wht, bp)


def kernel(user_ids, item_ids, gmf_user, gmf_item, mlp_user, mlp_item,
           W1, b1, W2, b2, Wp, bp):
    uids = user_ids.astype(jnp.int32)
    iids = item_ids.astype(jnp.int32)
    g, m = _sc_gather(uids, iids,
                      jnp.transpose(gmf_user), jnp.transpose(gmf_item),
                      jnp.transpose(mlp_user), jnp.transpose(mlp_item))
    W1t = jnp.transpose(W1)
    W2t = jnp.transpose(W2)
    Wgt = jnp.transpose(Wp[:D, :])
    Wht = jnp.transpose(Wp[D:, :])
    o = _tc_mlp(g, m, W1t, b1, W2t, b2, Wgt, Wht, bp)
    return jnp.squeeze(o, axis=0)


# speedup vs baseline: 1.3755x; 1.3755x over previous
import dataclasses
import functools

import jax
import jax.numpy as jnp
from jax import lax
from jax.experimental import pallas as pl
from jax.experimental.pallas import tpu as pltpu
from jax.experimental.pallas import tpu_sc as plsc

B = 16384
D = 16
NC, NS = 2, 16
NW = NC * NS
BPW = B // NW
NBUF = 8
LANES = 128
QUART = 128
NQ = BPW // QUART


def _sc_gather(su_ids, si_ids, pu2, pi2, gu_t, gi_t, mu_t, mi_t):
    mesh = plsc.VectorSubcoreMesh(
        core_axis_name="c", subcore_axis_name="s",
        num_cores=NC, num_subcores=NS)

    cp = pltpu.CompilerParams()
    if "needs_layout_passes" in pltpu.CompilerParams.__dataclass_fields__:
        cp = dataclasses.replace(cp, needs_layout_passes=False)

    ring_types = [pltpu.VMEM((D, LANES), jnp.float32)
                  for _ in range(4 * NBUF)]

    @functools.partial(
        pl.kernel,
        out_type=(jax.ShapeDtypeStruct((B, LANES), jnp.float32),
                  jax.ShapeDtypeStruct((B, LANES), jnp.float32)),
        mesh=mesh,
        compiler_params=cp,
        scratch_types=(
            [pltpu.SMEM((BPW,), jnp.int32),
             pltpu.SMEM((BPW,), jnp.int32),
             pltpu.VMEM((BPW,), jnp.int32),
             pltpu.VMEM((BPW,), jnp.int32),
             pltpu.VMEM((NQ, QUART), jnp.int32),
             pltpu.VMEM((NQ, QUART), jnp.int32),
             pltpu.VMEM((QUART, LANES), jnp.float32),
             pltpu.VMEM((QUART, LANES), jnp.float32)]
            + ring_types
            + [pltpu.SemaphoreType.DMA for _ in range(2 * NBUF)]
            + [pltpu.SemaphoreType.DMA]
        ),
    )
    def k(t_su, t_si, t_pu, t_pi, t_gu, t_gi, t_mu, t_mi,
          o_a, o_b, *scratch):
        su, si, vu, vi = scratch[0], scratch[1], scratch[2], scratch[3]
        puv, piv = scratch[4], scratch[5]
        st_a, st_b = scratch[6], scratch[7]
        rings = scratch[8:8 + 4 * NBUF]
        sems_u = scratch[8 + 4 * NBUF: 8 + 5 * NBUF]
        sems_i = scratch[8 + 5 * NBUF: 8 + 6 * NBUF]
        sem_s = scratch[8 + 6 * NBUF]

        wid = lax.axis_index("s") * NC + lax.axis_index("c")
        base = wid * BPW
        pltpu.sync_copy(t_su.at[pl.ds(base, BPW)], vu)
        pltpu.sync_copy(t_si.at[pl.ds(base, BPW)], vi)
        for q in range(NQ):
            off = pl.multiple_of(base + q * QUART, 8)
            pltpu.sync_copy(t_pu.at[pl.ds(off, QUART)], puv.at[q])
            pltpu.sync_copy(t_pi.at[pl.ds(off, QUART)], piv.at[q])

        iota16 = jax.lax.iota(jnp.int32, D)
        zero16 = jnp.zeros((D,), jnp.float32)

        def zrow(r, _):
            rv = jnp.full((D,), r, jnp.int32)
            for c in range(2 * D, LANES, D):
                plsc.store_scatter(st_a, [rv, iota16 + c], zero16)
                plsc.store_scatter(st_b, [rv, iota16 + c], zero16)
            return 0

        lax.fori_loop(0, QUART, zrow, 0)

        def spill(c, _):
            off = pl.multiple_of(c * 16, 8)
            uc = vu[pl.ds(off, 16)]
            ic = vi[pl.ds(off, 16)]
            for j in range(16):
                su[c * 16 + j] = jnp.sum(jnp.where(iota16 == j, uc, 0))
                si[c * 16 + j] = jnp.sum(jnp.where(iota16 == j, ic, 0))
            return 0

        lax.fori_loop(0, BPW // 16, spill, 0)

        def flag(sm, j):
            prev = sm[jnp.maximum(j - 1, 0)]
            return ((j % QUART) == 0) | ((sm[j] >> 7) != (prev >> 7))

        def fire_u(j, slot):
            ub = pl.multiple_of((su[j] >> 7) * LANES, LANES)
            pltpu.async_copy(t_gu.at[:, pl.ds(ub, LANES)],
                             rings[0 * NBUF + slot], sems_u[slot])
            pltpu.async_copy(t_mu.at[:, pl.ds(ub, LANES)],
                             rings[2 * NBUF + slot], sems_u[slot])

        def fire_i(j, slot):
            ib = pl.multiple_of((si[j] >> 7) * LANES, LANES)
            pltpu.async_copy(t_gi.at[:, pl.ds(ib, LANES)],
                             rings[1 * NBUF + slot], sems_i[slot])
            pltpu.async_copy(t_mi.at[:, pl.ds(ib, LANES)],
                             rings[3 * NBUF + slot], sems_i[slot])

        for j0 in range(NBUF):

            @pl.when(flag(su, j0))
            def _():
                fire_u(j0, j0)

            @pl.when(flag(si, j0))
            def _():
                fire_i(j0, j0)

        def extract_run(sm, st, ring_g, ring_m, j, qhi):
            tile = sm[j] >> 7

            def cond(k):
                kc = jnp.minimum(k, BPW - 1)
                return (k < qhi) & ((sm[kc] >> 7) == tile)

            def body(k):
                lane = jnp.full((D,), sm[k] & (LANES - 1), jnp.int32)
                row = jnp.full((D,), k % QUART, jnp.int32)
                ge = plsc.load_gather(ring_g, [iota16, lane])
                me = plsc.load_gather(ring_m, [iota16, lane])
                plsc.store_scatter(st, [row, iota16], ge)
                plsc.store_scatter(st, [row, iota16 + D], me)
                return k + 1

            lax.while_loop(cond, body, j)

        def step(j, slot, qhi):
            fu = flag(su, j)
            fi = flag(si, j)

            @pl.when(fu)
            def _():
                pltpu.make_async_copy(t_gu.at[:, pl.ds(0, LANES)],
                                      rings[0 * NBUF + slot],
                                      sems_u[slot]).wait()
                pltpu.make_async_copy(t_mu.at[:, pl.ds(0, LANES)],
                                      rings[2 * NBUF + slot],
                                      sems_u[slot]).wait()
                extract_run(su, st_a, rings[0 * NBUF + slot],
                            rings[2 * NBUF + slot], j, qhi)

            @pl.when(fi)
            def _():
                pltpu.make_async_copy(t_gi.at[:, pl.ds(0, LANES)],
                                      rings[1 * NBUF + slot],
                                      sems_i[slot]).wait()
                pltpu.make_async_copy(t_mi.at[:, pl.ds(0, LANES)],
                                      rings[3 * NBUF + slot],
                                      sems_i[slot]).wait()
                extract_run(si, st_b, rings[1 * NBUF + slot],
                            rings[3 * NBUF + slot], j, qhi)

            nxt = jnp.minimum(j + NBUF, BPW - 1)

            @pl.when((j + NBUF < BPW) & flag(su, nxt))
            def _():
                fire_u(nxt, slot)

            @pl.when((j + NBUF < BPW) & flag(si, nxt))
            def _():
                fire_i(nxt, slot)

        for q in range(NQ):
            qhi = (q + 1) * QUART

            def group(gi_, _, q=q, qhi=qhi):
                for slot in range(NBUF):
                    j = q * QUART + gi_ * NBUF + slot
                    step(j, slot, qhi)
                return 0

            lax.fori_loop(0, QUART // NBUF, group, 0)
            pltpu.async_copy(st_a, o_a.at[puv.at[q]], sem_s).wait()
            pltpu.async_copy(st_b, o_b.at[piv.at[q]], sem_s).wait()

    return k(su_ids, si_ids, pu2, pi2, gu_t, gi_t, mu_t, mi_t)


def _tc_mlp_body(xa, xb, w1a, w1b, b1, w2, b2, wg, wh, bpr, out_ref):
    f32 = jnp.float32
    a = xa[...]
    bv = xb[...]
    h1 = jnp.dot(a, w1a[...], preferred_element_type=f32)
    h1 = h1 + jnp.dot(bv, w1b[...], preferred_element_type=f32)
    h1 = jnp.maximum(h1 + b1[...], 0.0)
    h2 = jnp.dot(h1, w2[...], preferred_element_type=f32)
    h2 = jnp.maximum(h2 + b2[...], 0.0)
    o = jnp.dot(a * bv, wg[...], preferred_element_type=f32)
    o = o + jnp.dot(h2, wh[...], preferred_element_type=f32)
    out_ref[...] = o + bpr[...]


def _tc_mlp(xa, xb, W1a, W1b, b1, W2, b2, Wg, Wh, bp):
    return pl.pallas_call(
        _tc_mlp_body,
        out_shape=jax.ShapeDtypeStruct((B, 1), jnp.float32),
    )(xa, xb, W1a, W1b, b1, W2, b2, Wg, Wh, bp)


def kernel(user_ids, item_ids, gmf_user, gmf_item, mlp_user, mlp_item,
           W1, b1, W2, b2, Wp, bp):
    uids = user_ids.astype(jnp.int32)
    iids = item_ids.astype(jnp.int32)
    pu = jnp.argsort(uids).astype(jnp.int32)
    su_ids = jnp.take(uids, pu)
    pi = jnp.argsort(iids).astype(jnp.int32)
    si_ids = jnp.take(iids, pi)
    xa, xb = _sc_gather(su_ids, si_ids, pu, pi,
                        jnp.transpose(gmf_user), jnp.transpose(gmf_item),
                        jnp.transpose(mlp_user), jnp.transpose(mlp_item))
    zpad = jnp.zeros((LANES - 2 * D, W1.shape[1]), jnp.float32)
    zhead = jnp.zeros((D, W1.shape[1]), jnp.float32)
    W1a = jnp.concatenate([zhead, W1[:D, :], zpad], axis=0)
    W1b = jnp.concatenate([zhead, W1[D:, :], zpad], axis=0)
    Wg = jnp.concatenate([Wp[:D, :],
                          jnp.zeros((LANES - D, 1), jnp.float32)], axis=0)
    o = _tc_mlp(xa, xb, W1a, W1b, b1, W2, b2, Wg, Wp[D:, :], bp)
    return jnp.squeeze(o, axis=-1)


# speedup vs baseline: 1.5289x; 1.1115x over previous
import dataclasses
import functools

import jax
import jax.numpy as jnp
from jax import lax
from jax.experimental import pallas as pl
from jax.experimental.pallas import tpu as pltpu
from jax.experimental.pallas import tpu_sc as plsc

B = 16384
D = 16
NC, NS = 2, 16
NW = NC * NS
BPW = B // NW
NBUF = 8
LANES = 128
QUART = 128
NQ = BPW // QUART


def _sc_gather(su_ids, si_ids, pu2, pi2, gu_t, gi_t, mu_t, mi_t):
    mesh = plsc.VectorSubcoreMesh(
        core_axis_name="c", subcore_axis_name="s",
        num_cores=NC, num_subcores=NS)

    cp = pltpu.CompilerParams()
    if "needs_layout_passes" in pltpu.CompilerParams.__dataclass_fields__:
        cp = dataclasses.replace(cp, needs_layout_passes=False)

    ring_types = [pltpu.VMEM((D, LANES), jnp.float32)
                  for _ in range(4 * NBUF)]

    @functools.partial(
        pl.kernel,
        out_type=(jax.ShapeDtypeStruct((B, LANES), jnp.float32),
                  jax.ShapeDtypeStruct((B, LANES), jnp.float32)),
        mesh=mesh,
        compiler_params=cp,
        scratch_types=(
            [pltpu.SMEM((BPW,), jnp.int32),
             pltpu.SMEM((BPW,), jnp.int32),
             pltpu.VMEM((BPW,), jnp.int32),
             pltpu.VMEM((BPW,), jnp.int32),
             pltpu.VMEM((NQ, QUART), jnp.int32),
             pltpu.VMEM((NQ, QUART), jnp.int32),
             pltpu.VMEM((QUART, LANES), jnp.float32),
             pltpu.VMEM((QUART, LANES), jnp.float32)]
            + ring_types
            + [pltpu.SemaphoreType.DMA for _ in range(2 * NBUF)]
            + [pltpu.SemaphoreType.DMA]
        ),
    )
    def k(t_su, t_si, t_pu, t_pi, t_gu, t_gi, t_mu, t_mi,
          o_a, o_b, *scratch):
        su, si, vu, vi = scratch[0], scratch[1], scratch[2], scratch[3]
        puv, piv = scratch[4], scratch[5]
        st_a, st_b = scratch[6], scratch[7]
        rings = scratch[8:8 + 4 * NBUF]
        sems_u = scratch[8 + 4 * NBUF: 8 + 5 * NBUF]
        sems_i = scratch[8 + 5 * NBUF: 8 + 6 * NBUF]
        sem_s = scratch[8 + 6 * NBUF]

        wid = lax.axis_index("s") * NC + lax.axis_index("c")
        base = wid * BPW
        pltpu.sync_copy(t_su.at[pl.ds(base, BPW)], vu)
        pltpu.sync_copy(t_si.at[pl.ds(base, BPW)], vi)
        for q in range(NQ):
            off = pl.multiple_of(base + q * QUART, 8)
            pltpu.sync_copy(t_pu.at[pl.ds(off, QUART)], puv.at[q])
            pltpu.sync_copy(t_pi.at[pl.ds(off, QUART)], piv.at[q])

        iota16 = jax.lax.iota(jnp.int32, D)
        zero16 = jnp.zeros((D,), jnp.float32)

        def spill(c, _):
            off = pl.multiple_of(c * 16, 8)
            uc = vu[pl.ds(off, 16)]
            ic = vi[pl.ds(off, 16)]
            for j in range(16):
                su[c * 16 + j] = jnp.sum(jnp.where(iota16 == j, uc, 0))
                si[c * 16 + j] = jnp.sum(jnp.where(iota16 == j, ic, 0))
            return 0

        lax.fori_loop(0, 1, spill, 0)

        def flag(sm, j):
            prev = sm[jnp.maximum(j - 1, 0)]
            return ((j % QUART) == 0) | ((sm[j] >> 7) != (prev >> 7))

        def fire_u(j, slot):
            ub = pl.multiple_of((su[j] >> 7) * LANES, LANES)
            pltpu.async_copy(t_gu.at[:, pl.ds(ub, LANES)],
                             rings[0 * NBUF + slot], sems_u[slot])
            pltpu.async_copy(t_mu.at[:, pl.ds(ub, LANES)],
                             rings[2 * NBUF + slot], sems_u[slot])

        def fire_i(j, slot):
            ib = pl.multiple_of((si[j] >> 7) * LANES, LANES)
            pltpu.async_copy(t_gi.at[:, pl.ds(ib, LANES)],
                             rings[1 * NBUF + slot], sems_i[slot])
            pltpu.async_copy(t_mi.at[:, pl.ds(ib, LANES)],
                             rings[3 * NBUF + slot], sems_i[slot])

        for j0 in range(NBUF):

            @pl.when(flag(su, j0))
            def _():
                fire_u(j0, j0)

            @pl.when(flag(si, j0))
            def _():
                fire_i(j0, j0)

        lax.fori_loop(1, BPW // 16, spill, 0)

        def zrow(r, _):
            rv = jnp.full((D,), r, jnp.int32)
            for c in range(2 * D, LANES, D):
                plsc.store_scatter(st_a, [rv, iota16 + c], zero16)
                plsc.store_scatter(st_b, [rv, iota16 + c], zero16)
            return 0

        lax.fori_loop(0, QUART, zrow, 0)

        def extract_run(sm, st, ring_g, ring_m, j, qhi):
            tile = sm[j] >> 7

            def cond(k):
                kc = jnp.minimum(k, BPW - 1)
                return (k < qhi) & ((sm[kc] >> 7) == tile)

            def body(k):
                lane = jnp.full((D,), sm[k] & (LANES - 1), jnp.int32)
                row = jnp.full((D,), k % QUART, jnp.int32)
                ge = plsc.load_gather(ring_g, [iota16, lane])
                me = plsc.load_gather(ring_m, [iota16, lane])
                plsc.store_scatter(st, [row, iota16], ge)
                plsc.store_scatter(st, [row, iota16 + D], me)
                return k + 1

            lax.while_loop(cond, body, j)

        def step(j, slot, qhi):
            fu = flag(su, j)
            fi = flag(si, j)

            @pl.when(fu)
            def _():
                pltpu.make_async_copy(t_gu.at[:, pl.ds(0, LANES)],
                                      rings[0 * NBUF + slot],
                                      sems_u[slot]).wait()
                pltpu.make_async_copy(t_mu.at[:, pl.ds(0, LANES)],
                                      rings[2 * NBUF + slot],
                                      sems_u[slot]).wait()
                extract_run(su, st_a, rings[0 * NBUF + slot],
                            rings[2 * NBUF + slot], j, qhi)

            @pl.when(fi)
            def _():
                pltpu.make_async_copy(t_gi.at[:, pl.ds(0, LANES)],
                                      rings[1 * NBUF + slot],
                                      sems_i[slot]).wait()
                pltpu.make_async_copy(t_mi.at[:, pl.ds(0, LANES)],
                                      rings[3 * NBUF + slot],
                                      sems_i[slot]).wait()
                extract_run(si, st_b, rings[1 * NBUF + slot],
                            rings[3 * NBUF + slot], j, qhi)

            nxt = jnp.minimum(j + NBUF, BPW - 1)

            @pl.when((j + NBUF < BPW) & flag(su, nxt))
            def _():
                fire_u(nxt, slot)

            @pl.when((j + NBUF < BPW) & flag(si, nxt))
            def _():
                fire_i(nxt, slot)

        for q in range(NQ):
            qhi = (q + 1) * QUART

            def group(gi_, _, q=q, qhi=qhi):
                for slot in range(NBUF):
                    j = q * QUART + gi_ * NBUF + slot
                    step(j, slot, qhi)
                return 0

            lax.fori_loop(0, QUART // NBUF, group, 0)
            pltpu.async_copy(st_a, o_a.at[puv.at[q]], sem_s).wait()
            pltpu.async_copy(st_b, o_b.at[piv.at[q]], sem_s).wait()

    return k(su_ids, si_ids, pu2, pi2, gu_t, gi_t, mu_t, mi_t)


def _tc_mlp_body(xa, xb, w1a, w1b, b1, w2, b2, wg, wh, bpr, out_ref):
    f32 = jnp.float32
    a = xa[...]
    bv = xb[...]
    h1 = jnp.dot(a, w1a[...], preferred_element_type=f32)
    h1 = h1 + jnp.dot(bv, w1b[...], preferred_element_type=f32)
    h1 = jnp.maximum(h1 + b1[...], 0.0)
    h2 = jnp.dot(h1, w2[...], preferred_element_type=f32)
    h2 = jnp.maximum(h2 + b2[...], 0.0)
    o = jnp.dot(a * bv, wg[...], preferred_element_type=f32)
    o = o + jnp.dot(h2, wh[...], preferred_element_type=f32)
    out_ref[...] = o + bpr[...]


def _tc_mlp(xa, xb, W1a, W1b, b1, W2, b2, Wg, Wh, bp):
    return pl.pallas_call(
        _tc_mlp_body,
        out_shape=jax.ShapeDtypeStruct((B, 1), jnp.float32),
    )(xa, xb, W1a, W1b, b1, W2, b2, Wg, Wh, bp)


def kernel(user_ids, item_ids, gmf_user, gmf_item, mlp_user, mlp_item,
           W1, b1, W2, b2, Wp, bp):
    uids = user_ids.astype(jnp.int32)
    iids = item_ids.astype(jnp.int32)
    iota_b = jax.lax.iota(jnp.int32, B)
    su_ids, pu = jax.lax.sort((uids, iota_b), num_keys=1)
    si_ids, pi = jax.lax.sort((iids, iota_b), num_keys=1)
    xa, xb = _sc_gather(su_ids, si_ids, pu, pi,
                        jnp.transpose(gmf_user), jnp.transpose(gmf_item),
                        jnp.transpose(mlp_user), jnp.transpose(mlp_item))
    zpad = jnp.zeros((LANES - 2 * D, W1.shape[1]), jnp.float32)
    zhead = jnp.zeros((D, W1.shape[1]), jnp.float32)
    W1a = jnp.concatenate([zhead, W1[:D, :], zpad], axis=0)
    W1b = jnp.concatenate([zhead, W1[D:, :], zpad], axis=0)
    Wg = jnp.concatenate([Wp[:D, :],
                          jnp.zeros((LANES - D, 1), jnp.float32)], axis=0)
    o = _tc_mlp(xa, xb, W1a, W1b, b1, W2, b2, Wg, Wp[D:, :], bp)
    return jnp.squeeze(o, axis=-1)
